# Initial kernel scaffold; baseline (speedup 1.0000x reference)
import jax
import jax.numpy as jnp
from jax.experimental import pallas as pl


def kernel(x, Wg, bg, We, be):
    raise NotImplementedError("write your pallas kernel here")


# speedup vs baseline: 3.6111x; 3.6111x over previous
import jax
import jax.numpy as jnp
from jax.experimental import pallas as pl
from jax.experimental.pallas import tpu as pltpu
from functools import partial


def _moe_body(x_ref, wg_ref, bg_ref, we_ref, be_ref, out_ref, *, n_experts):
    xb = x_ref[...]
    logits = jnp.dot(xb, wg_ref[...], preferred_element_type=jnp.float32)
    logits = logits + bg_ref[...]
    bt = logits.shape[0]
    idx = jax.lax.broadcasted_iota(jnp.int32, (bt, n_experts), 1)
    m1 = jnp.max(logits, axis=-1, keepdims=True)
    a1 = jnp.min(jnp.where(logits == m1, idx, n_experts), axis=-1, keepdims=True)
    sel1 = idx == a1
    masked = jnp.where(sel1, -jnp.inf, logits)
    m2 = jnp.max(masked, axis=-1, keepdims=True)
    a2 = jnp.min(jnp.where(masked == m2, idx, n_experts), axis=-1, keepdims=True)
    sel2 = idx == a2
    t = jnp.exp(m2 - m1)
    w1 = 1.0 / (1.0 + t)
    w = jnp.where(sel1, w1, 0.0) + jnp.where(sel2, t * w1, 0.0)
    xb_bf = xb.astype(jnp.bfloat16)
    acc = jnp.dot(w, be_ref[...], preferred_element_type=jnp.float32)
    for e in range(n_experts):
        ye = jnp.dot(xb_bf, we_ref[e], preferred_element_type=jnp.float32)
        acc = acc + w[:, e:e + 1] * ye
    out_ref[...] = acc


def kernel(x, Wg, bg, We, be):
    B, D = x.shape
    E, _, O = We.shape
    BT = 512
    nb = B // BT
    we_bf = We.astype(jnp.bfloat16)
    bg2 = bg.reshape(1, E)
    body = partial(_moe_body, n_experts=E)
    return pl.pallas_call(
        body,
        grid=(nb,),
        in_specs=[
            pl.BlockSpec((BT, D), lambda i: (i, 0)),
            pl.BlockSpec((D, E), lambda i: (0, 0)),
            pl.BlockSpec((1, E), lambda i: (0, 0)),
            pl.BlockSpec((E, D, O), lambda i: (0, 0, 0)),
            pl.BlockSpec((E, O), lambda i: (0, 0)),
        ],
        out_specs=pl.BlockSpec((BT, O), lambda i: (i, 0)),
        out_shape=jax.ShapeDtypeStruct((B, O), jnp.float32),
        compiler_params=pltpu.CompilerParams(
            dimension_semantics=("arbitrary",),
        ),
    )(x, Wg, bg2, we_bf, be)
